# baseline (device time: 168911 ns/iter reference)
import jax
import jax.numpy as jnp
from jax import lax
from jax.experimental import pallas as pl
from jax.experimental.pallas import tpu as pltpu

N_DEV = 4
N_HOP = N_DEV - 1
SEG = 2


def kernel(x, w_mat):
    m, _ = x.shape
    _, n = w_mat.shape
    m_chunk = m // N_DEV
    m_seg = m_chunk // SEG
    n_half = n // 2

    w = w_mat.astype(jnp.bfloat16)

    def body(x_ref, w_ref, out_ref, comm, z_buf, ag_buf,
             rs_send_sems, rs_recv_sems, ag_send_sems, ag_recv_sems,
             copy_sems):
        my = lax.axis_index("i")
        right = lax.rem(my + 1, N_DEV)
        left = lax.rem(my + N_DEV - 1, N_DEV)
        dev = (right, left)
        col = (pl.ds(0, n_half), pl.ds(n_half, n_half))

        barrier_sem = pltpu.get_barrier_semaphore()
        for nbr in (left, right):
            pl.semaphore_signal(
                barrier_sem, inc=1,
                device_id=(nbr,), device_id_type=pl.DeviceIdType.MESH,
            )
        pl.semaphore_wait(barrier_sem, 2)

        def seg_dot(c, k, d):
            xs = x_ref[pl.ds(c * m_chunk + k * m_seg, m_seg), :]
            ws = w_ref[:, col[d]]
            return lax.dot_general(
                xs.astype(jnp.bfloat16), ws, (((1,), (0,)), ((), ())),
                preferred_element_type=jnp.float32,
            )

        def out_seg(c, k, d):
            return out_ref.at[pl.ds(c * m_chunk + k * m_seg, m_seg), col[d]]

        def rs_chunk(d, s):
            return lax.rem(my - s + N_DEV, N_DEV) if d == 0 \
                else lax.rem(my + s, N_DEV)

        sends = []
        rs_rdma = {}
        ag_rdma = {}

        def rs_send(d, s, k, src):
            r = pltpu.make_async_remote_copy(
                src_ref=src,
                dst_ref=comm.at[d, s, k],
                send_sem=rs_send_sems.at[d, s, k],
                recv_sem=rs_recv_sems.at[d, s, k],
                device_id=(dev[d],),
                device_id_type=pl.DeviceIdType.MESH,
            )
            r.start()
            rs_rdma[(d, s, k)] = r
            sends.append(r)

        for k in range(SEG):
            for d in range(2):
                comm[d, N_HOP, k] = seg_dot(my, k, d).astype(jnp.bfloat16)
                rs_send(d, 0, k, comm.at[d, N_HOP, k])

        p = {}
        for k in range(SEG):
            for d in range(2):
                p[(d, k)] = seg_dot(rs_chunk(d, 1), k, d)

        own = (lax.rem(my + 1, N_DEV), lax.rem(my + N_DEV - 1, N_DEV))
        for s in range(1, N_HOP):
            for k in range(SEG):
                for d in range(2):
                    rs_rdma[(d, s - 1, k)].wait_recv()
                    acc = p[(d, k)] + comm[d, s - 1, k].astype(jnp.float32)
                    comm[d, s - 1, k] = acc.astype(jnp.bfloat16)
                    rs_send(d, s, k, comm.at[d, s - 1, k])
            for k in range(SEG):
                for d in range(2):
                    c_next = rs_chunk(d, s + 1) if s + 1 < N_HOP else own[d]
                    p[(d, k)] = seg_dot(c_next, k, d)

        copies = []
        for k in range(SEG):
            for d in range(2):
                rs_rdma[(d, N_HOP - 1, k)].wait_recv()
                full = p[(d, k)] + comm[d, N_HOP - 1, k].astype(jnp.float32)
                z_buf[d, k] = (full * jax.nn.sigmoid(full)).astype(jnp.bfloat16)
                cp = pltpu.make_async_copy(
                    z_buf.at[d, k], out_seg(own[d], k, d),
                    copy_sems.at[d, 0, k])
                cp.start()
                copies.append(cp)
                r = pltpu.make_async_remote_copy(
                    src_ref=z_buf.at[d, k],
                    dst_ref=ag_buf.at[d, 0, k],
                    send_sem=ag_send_sems.at[d, 0, k],
                    recv_sem=ag_recv_sems.at[d, 0, k],
                    device_id=(dev[d],),
                    device_id_type=pl.DeviceIdType.MESH,
                )
                r.start()
                ag_rdma[(d, 0, k)] = r
                sends.append(r)

        def ag_recv_chunk(d, g):
            return lax.rem(my - g + N_DEV, N_DEV) if d == 0 \
                else lax.rem(my + g, N_DEV)

        for g in range(1, N_HOP):
            for k in range(SEG):
                for d in range(2):
                    ag_rdma[(d, g - 1, k)].wait_recv()
                    cp = pltpu.make_async_copy(
                        ag_buf.at[d, g - 1, k],
                        out_seg(ag_recv_chunk(d, g - 1), k, d),
                        copy_sems.at[d, g, k])
                    cp.start()
                    copies.append(cp)
                    last = g == N_HOP - 1
                    r = pltpu.make_async_remote_copy(
                        src_ref=ag_buf.at[d, g - 1, k],
                        dst_ref=(out_seg(ag_recv_chunk(d, g - 1), k, d) if last
                                 else ag_buf.at[d, g, k]),
                        send_sem=ag_send_sems.at[d, g, k],
                        recv_sem=ag_recv_sems.at[d, g, k],
                        device_id=(dev[d],),
                        device_id_type=pl.DeviceIdType.MESH,
                    )
                    r.start()
                    ag_rdma[(d, g, k)] = r
                    sends.append(r)

        for k in range(SEG):
            for d in range(2):
                ag_rdma[(d, N_HOP - 1, k)].wait_recv()
        for cp in copies:
            cp.wait()
        for r in sends:
            r.wait_send()

    return pl.pallas_call(
        body,
        out_shape=jax.ShapeDtypeStruct((m, n), jnp.bfloat16),
        in_specs=[
            pl.BlockSpec(memory_space=pltpu.VMEM),
            pl.BlockSpec(memory_space=pltpu.VMEM),
        ],
        out_specs=pl.BlockSpec(memory_space=pl.ANY),
        scratch_shapes=[
            pltpu.VMEM((2, N_HOP + 1, SEG, m_seg, n_half), jnp.bfloat16),
            pltpu.VMEM((2, SEG, m_seg, n_half), jnp.bfloat16),
            pltpu.VMEM((2, N_HOP - 1, SEG, m_seg, n_half), jnp.bfloat16),
            pltpu.SemaphoreType.DMA((2, N_HOP, SEG)),
            pltpu.SemaphoreType.DMA((2, N_HOP, SEG)),
            pltpu.SemaphoreType.DMA((2, N_HOP, SEG)),
            pltpu.SemaphoreType.DMA((2, N_HOP, SEG)),
            pltpu.SemaphoreType.DMA((2, N_HOP, SEG)),
        ],
        compiler_params=pltpu.CompilerParams(
            collective_id=0,
            vmem_limit_bytes=100 * 1024 * 1024,
        ),
    )(x, w)


# device time: 167756 ns/iter; 1.0069x vs baseline; 1.0069x over previous
import jax
import jax.numpy as jnp
from jax import lax
from jax.experimental import pallas as pl
from jax.experimental.pallas import tpu as pltpu

N_DEV = 4
N_HOP = N_DEV - 1
SEG = 2
COMPUTE = True


def kernel(x, w_mat):
    m, _ = x.shape
    _, n = w_mat.shape
    m_chunk = m // N_DEV
    m_seg = m_chunk // SEG
    n_half = n // 2

    w = w_mat.astype(jnp.bfloat16)

    def body(x_ref, w_ref, out_ref, comm0, comm1, z_buf, ag_buf,
             rs_send_sems, rs_recv_sems, ag_send_sems, ag_recv_sems,
             copy_sems):
        comm_d = (comm0, comm1)
        my = lax.axis_index("i")
        right = lax.rem(my + 1, N_DEV)
        left = lax.rem(my + N_DEV - 1, N_DEV)
        dev = (right, left)
        col = (pl.ds(0, n_half), pl.ds(n_half, n_half))

        barrier_sem = pltpu.get_barrier_semaphore()
        for nbr in (left, right):
            pl.semaphore_signal(
                barrier_sem, inc=1,
                device_id=(nbr,), device_id_type=pl.DeviceIdType.MESH,
            )
        pl.semaphore_wait(barrier_sem, 2)

        def seg_dot(c, k, d):
            xs = x_ref[pl.ds(c * m_chunk + k * m_seg, m_seg), :]
            ws = w_ref[:, col[d]]
            return lax.dot_general(
                xs.astype(jnp.bfloat16), ws, (((1,), (0,)), ((), ())),
                preferred_element_type=jnp.float32,
            )

        def out_seg(c, k, d):
            return out_ref.at[pl.ds(c * m_chunk + k * m_seg, m_seg), col[d]]

        def rs_chunk(d, s):
            return lax.rem(my - s + N_DEV, N_DEV) if d == 0 \
                else lax.rem(my + s, N_DEV)

        sends = []
        rs_rdma = {}
        ag_rdma = {}

        def rs_send(d, s, k, src):
            r = pltpu.make_async_remote_copy(
                src_ref=src,
                dst_ref=comm_d[d].at[s, k],
                send_sem=rs_send_sems.at[d, s, k],
                recv_sem=rs_recv_sems.at[d, s, k],
                device_id=(dev[d],),
                device_id_type=pl.DeviceIdType.MESH,
            )
            r.start()
            rs_rdma[(d, s, k)] = r
            sends.append(r)

        for k in range(SEG):
            for d in range(2):
                if COMPUTE:
                    comm_d[d][N_HOP, k] = seg_dot(my, k, d).astype(jnp.bfloat16)
                rs_send(d, 0, k, comm_d[d].at[N_HOP, k])

        p = {}
        if COMPUTE:
            for k in range(SEG):
                for d in range(2):
                    p[(d, k)] = seg_dot(rs_chunk(d, 1), k, d)

        own = (lax.rem(my + 1, N_DEV), lax.rem(my + N_DEV - 1, N_DEV))
        for s in range(1, N_HOP):
            for k in range(SEG):
                for d in range(2):
                    rs_rdma[(d, s - 1, k)].wait_recv()
                    if COMPUTE:
                        acc = p[(d, k)] + comm_d[d][s - 1, k].astype(jnp.float32)
                        comm_d[d][s - 1, k] = acc.astype(jnp.bfloat16)
                    rs_send(d, s, k, comm_d[d].at[s - 1, k])
            if COMPUTE:
                for k in range(SEG):
                    for d in range(2):
                        c_next = rs_chunk(d, s + 1) if s + 1 < N_HOP else own[d]
                        p[(d, k)] = seg_dot(c_next, k, d)

        copies = []
        for k in range(SEG):
            for d in range(2):
                rs_rdma[(d, N_HOP - 1, k)].wait_recv()
                if COMPUTE:
                    full = p[(d, k)] \
                        + comm_d[d][N_HOP - 1, k].astype(jnp.float32)
                    z_buf[d, k] = (
                        full * jax.nn.sigmoid(full)).astype(jnp.bfloat16)
                cp = pltpu.make_async_copy(
                    z_buf.at[d, k], out_seg(own[d], k, d),
                    copy_sems.at[d, 0, k])
                cp.start()
                copies.append(cp)
                r = pltpu.make_async_remote_copy(
                    src_ref=z_buf.at[d, k],
                    dst_ref=ag_buf.at[d, 0, k],
                    send_sem=ag_send_sems.at[d, 0, k],
                    recv_sem=ag_recv_sems.at[d, 0, k],
                    device_id=(dev[d],),
                    device_id_type=pl.DeviceIdType.MESH,
                )
                r.start()
                ag_rdma[(d, 0, k)] = r
                sends.append(r)

        def ag_recv_chunk(d, g):
            return lax.rem(my - g + N_DEV, N_DEV) if d == 0 \
                else lax.rem(my + g, N_DEV)

        for g in range(1, N_HOP):
            for k in range(SEG):
                for d in range(2):
                    ag_rdma[(d, g - 1, k)].wait_recv()
                    cp = pltpu.make_async_copy(
                        ag_buf.at[d, g - 1, k],
                        out_seg(ag_recv_chunk(d, g - 1), k, d),
                        copy_sems.at[d, g, k])
                    cp.start()
                    copies.append(cp)
                    last = g == N_HOP - 1
                    r = pltpu.make_async_remote_copy(
                        src_ref=ag_buf.at[d, g - 1, k],
                        dst_ref=(out_seg(ag_recv_chunk(d, g - 1), k, d) if last
                                 else ag_buf.at[d, g, k]),
                        send_sem=ag_send_sems.at[d, g, k],
                        recv_sem=ag_recv_sems.at[d, g, k],
                        device_id=(dev[d],),
                        device_id_type=pl.DeviceIdType.MESH,
                    )
                    r.start()
                    ag_rdma[(d, g, k)] = r
                    sends.append(r)

        for k in range(SEG):
            for d in range(2):
                ag_rdma[(d, N_HOP - 1, k)].wait_recv()
        for cp in copies:
            cp.wait()
        for r in sends:
            r.wait_send()

    return pl.pallas_call(
        body,
        out_shape=jax.ShapeDtypeStruct((m, n), jnp.bfloat16),
        in_specs=[
            pl.BlockSpec(memory_space=pltpu.VMEM),
            pl.BlockSpec(memory_space=pltpu.VMEM),
        ],
        out_specs=pl.BlockSpec(memory_space=pl.ANY),
        scratch_shapes=[
            pltpu.VMEM((N_HOP + 1, SEG, m_seg, n_half), jnp.bfloat16),
            pltpu.VMEM((N_HOP + 1, SEG, m_seg, n_half), jnp.bfloat16),
            pltpu.VMEM((2, SEG, m_seg, n_half), jnp.bfloat16),
            pltpu.VMEM((2, N_HOP - 1, SEG, m_seg, n_half), jnp.bfloat16),
            pltpu.SemaphoreType.DMA((2, N_HOP, SEG)),
            pltpu.SemaphoreType.DMA((2, N_HOP, SEG)),
            pltpu.SemaphoreType.DMA((2, N_HOP, SEG)),
            pltpu.SemaphoreType.DMA((2, N_HOP, SEG)),
            pltpu.SemaphoreType.DMA((2, N_HOP, SEG)),
        ],
        compiler_params=pltpu.CompilerParams(
            collective_id=0,
            vmem_limit_bytes=100 * 1024 * 1024,
        ),
    )(x, w)
